# baseline (speedup 1.0000x reference)
import functools

import jax
import jax.numpy as jnp
from jax import lax
from jax.experimental import pallas as pl
from jax.experimental.pallas import tpu as pltpu
from jax.experimental.pallas import tpu_sc as plsc

RADIUS = 0.4
NUM_BASIS = 10
C_IN = 32
C_OUT = 32
GAMMA = (NUM_BASIS - 1) / RADIUS

N = 2048
BATCH = 2
NCORES = 2
NSUB = 16
LANES = 16
PTS = N // NSUB
NBLK = N // LANES
KPAD = 12
ROW = KPAD * C_IN
R2 = RADIUS * RADIUS


P = 4
HALF = PTS // 2


def _sc_body(geom, featT, acc_out, xv, yv, zv, fv, nb0, nb1, nb2, nb3, ubuf,
             stage):
    b = lax.axis_index("c")
    s = lax.axis_index("s")
    gbase = b * (3 * N)
    pltpu.sync_copy(geom.at[pl.ds(gbase, N)], xv)
    pltpu.sync_copy(geom.at[pl.ds(gbase + N, N)], yv)
    pltpu.sync_copy(geom.at[pl.ds(gbase + 2 * N, N)], zv)
    pltpu.sync_copy(featT.at[pl.ds(b * (N * C_IN), N * C_IN)], fv)

    lane_i = lax.iota(jnp.int32, 16)
    kvec = lane_i.astype(jnp.float32)
    zero16 = jnp.zeros((16,), jnp.float32)
    zero16i = jnp.zeros((16,), jnp.int32)
    nbufs = (nb0, nb1, nb2, nb3)

    def zinit(i, _):
        for nb in nbufs:
            nb[pl.ds(i * LANES, LANES)] = zero16i
        return 0

    lax.fori_loop(0, (N + LANES) // LANES, zinit, 0)

    def pass_body(ipass, _):
        p0 = s * PTS + ipass * P
        xi, yi, zi = [], [], []
        for q in range(P):
            pidx = jnp.full((LANES,), p0 + q, jnp.int32)
            xi.append(plsc.load_gather(xv, [pidx]))
            yi.append(plsc.load_gather(yv, [pidx]))
            zi.append(plsc.load_gather(zv, [pidx]))

        def blk(jb, offs):
            base = jb * LANES
            xb = xv[pl.ds(base, LANES)]
            yb = yv[pl.ds(base, LANES)]
            zb = zv[pl.ds(base, LANES)]
            jv = base + lane_i
            new_offs = []
            for q in range(P):
                dx = xb - xi[q]
                dy = yb - yi[q]
                dz = zb - zi[q]
                d2 = dx * dx + dy * dy + dz * dz
                m = d2 < R2
                incl = plsc.cumsum(m.astype(jnp.int32))
                idx = offs[q] + incl
                plsc.store_scatter(nbufs[q], [idx], jv, mask=m)
                new_offs.append(offs[q] + incl[LANES - 1])
            return tuple(new_offs)

        moff = jnp.full((LANES,), -1, jnp.int32)
        cntv = lax.fori_loop(0, NBLK, blk, (moff,) * P)
        cnts = [cntv[q][0] + 1 for q in range(P)]

        for q in range(P):
            ip = ipass * P + q
            nbq = nbufs[q]
            cnt = cnts[q]

            def b1(mb, _):
                o = mb * LANES
                jvq = nbq[pl.ds(o, LANES)]
                dx = plsc.load_gather(xv, [jvq]) - xi[q]
                dy = plsc.load_gather(yv, [jvq]) - yi[q]
                dz = plsc.load_gather(zv, [jvq]) - zi[q]
                d2v = dx * dx + dy * dy + dz * dz + 1e-12
                iv = plsc.bitcast(d2v, jnp.int32)
                iv = 0x5F3759DF - (iv >> 1)
                z = plsc.bitcast(iv, jnp.float32)
                z = z * (1.5 - 0.5 * d2v * z * z)
                z = z * (1.5 - 0.5 * d2v * z * z)
                z = z * (1.5 - 0.5 * d2v * z * z)
                ubuf[pl.ds(o, LANES)] = GAMMA * (d2v * z)
                return 0

            lax.fori_loop(0, (cnt + LANES - 1) // LANES, b1, 0)

            def b2(m, accs):
                j = nbq[pl.ds(m, LANES)][0]
                u = ubuf[pl.ds(m, LANES)][0]
                e = u - kvec
                rb = jnp.exp(-(e * e))
                fidx = j + lane_i * N
                f0 = plsc.load_gather(fv, [fidx])
                f1 = plsc.load_gather(fv, [fidx + 16 * N])
                out = []
                for k in range(NUM_BASIS):
                    w = rb[k]
                    out.append(accs[2 * k] + w * f0)
                    out.append(accs[2 * k + 1] + w * f1)
                return tuple(out)

            accs = lax.fori_loop(0, cnt, b2, (zero16,) * (2 * NUM_BASIS))

            rowbase = (ip % HALF) * ROW
            for k in range(NUM_BASIS):
                stage[pl.ds(rowbase + 32 * k, 16)] = accs[2 * k]
                stage[pl.ds(rowbase + 32 * k + 16, 16)] = accs[2 * k + 1]
            for k in range(NUM_BASIS, KPAD):
                stage[pl.ds(rowbase + 32 * k, 16)] = zero16
                stage[pl.ds(rowbase + 32 * k + 16, 16)] = zero16
        return 0

    out_base = b * (N * ROW) + s * PTS * ROW
    lax.fori_loop(0, HALF // P, pass_body, 0)
    pltpu.sync_copy(stage, acc_out.at[pl.ds(out_base, HALF * ROW)])
    lax.fori_loop(HALF // P, PTS // P, pass_body, 0)
    pltpu.sync_copy(
        stage, acc_out.at[pl.ds(out_base + HALF * ROW, HALF * ROW)])


def _tc_body(acc_ref, wt_ref, out_ref):
    out_ref[0] = lax.dot_general(
        wt_ref[...], acc_ref[0],
        (((1,), (1,)), ((), ())),
        preferred_element_type=jnp.float32,
    )


def kernel(features, geometry, W):
    featF = features.reshape(BATCH * C_IN * N)
    geomT = jnp.transpose(geometry, (0, 2, 1)).reshape(BATCH * 3 * N)
    wt = jnp.transpose(W, (1, 0, 2)).reshape(C_OUT, NUM_BASIS * C_IN)
    wt = jnp.pad(wt, ((0, 0), (0, (KPAD - NUM_BASIS) * C_IN)))

    mesh = plsc.VectorSubcoreMesh(core_axis_name="c", subcore_axis_name="s")
    sc = pl.kernel(
        _sc_body,
        mesh=mesh,
        compiler_params=pltpu.CompilerParams(needs_layout_passes=False),
        out_type=jax.ShapeDtypeStruct((BATCH * N * ROW,), jnp.float32),
        scratch_types=[
            pltpu.VMEM((N,), jnp.float32),
            pltpu.VMEM((N,), jnp.float32),
            pltpu.VMEM((N,), jnp.float32),
            pltpu.VMEM((N * C_IN,), jnp.float32),
            pltpu.VMEM((N + LANES,), jnp.int32),
            pltpu.VMEM((N + LANES,), jnp.int32),
            pltpu.VMEM((N + LANES,), jnp.int32),
            pltpu.VMEM((N + LANES,), jnp.int32),
            pltpu.VMEM((N + LANES,), jnp.float32),
            pltpu.VMEM((HALF * ROW,), jnp.float32),
        ],
    )
    acc = sc(geomT, featF).reshape(BATCH, N, ROW)

    nt = 8
    out = pl.pallas_call(
        _tc_body,
        grid=(BATCH, nt),
        in_specs=[
            pl.BlockSpec((1, N // nt, ROW), lambda b, t: (b, t, 0)),
            pl.BlockSpec((C_OUT, ROW), lambda b, t: (0, 0)),
        ],
        out_specs=pl.BlockSpec((1, C_OUT, N // nt), lambda b, t: (b, 0, t)),
        out_shape=jax.ShapeDtypeStruct((BATCH, C_OUT, N), jnp.float32),
    )(acc, wt)
    return out


# speedup vs baseline: 1.1598x; 1.1598x over previous
import functools

import jax
import jax.numpy as jnp
from jax import lax
from jax.experimental import pallas as pl
from jax.experimental.pallas import tpu as pltpu
from jax.experimental.pallas import tpu_sc as plsc

RADIUS = 0.4
NUM_BASIS = 10
C_IN = 32
C_OUT = 32
GAMMA = (NUM_BASIS - 1) / RADIUS

N = 2048
BATCH = 2
NCORES = 2
NSUB = 16
LANES = 16
PTS = N // NSUB
NBLK = N // LANES
KPAD = 12
ROW = KPAD * C_IN
R2 = RADIUS * RADIUS


P = 4
HALF = PTS // 2


def _sc_body(geom, featT, acc_out, xv, yv, zv, fv, nb0, nb1, nb2, nb3, ubuf,
             stage):
    b = lax.axis_index("c")
    s = lax.axis_index("s")
    gbase = b * (3 * N)
    pltpu.sync_copy(geom.at[pl.ds(gbase, N)], xv)
    pltpu.sync_copy(geom.at[pl.ds(gbase + N, N)], yv)
    pltpu.sync_copy(geom.at[pl.ds(gbase + 2 * N, N)], zv)
    pltpu.sync_copy(featT.at[pl.ds(b * (N * C_IN), N * C_IN)], fv)

    lane_i = lax.iota(jnp.int32, 16)
    kvec = lane_i.astype(jnp.float32)
    zero16 = jnp.zeros((16,), jnp.float32)
    zero16i = jnp.zeros((16,), jnp.int32)
    nbufs = (nb0, nb1, nb2, nb3)

    def zinit(i, _):
        for nb in nbufs:
            nb[pl.ds(i * LANES, LANES)] = zero16i
        return 0

    lax.fori_loop(0, (N + LANES) // LANES, zinit, 0)

    def pass_body(ipass, _):
        p0 = s * PTS + ipass * P
        xi, yi, zi = [], [], []
        for q in range(P):
            pidx = jnp.full((LANES,), p0 + q, jnp.int32)
            xi.append(plsc.load_gather(xv, [pidx]))
            yi.append(plsc.load_gather(yv, [pidx]))
            zi.append(plsc.load_gather(zv, [pidx]))

        def blk(jb, offs):
            base = jb * LANES
            xb = xv[pl.ds(base, LANES)]
            yb = yv[pl.ds(base, LANES)]
            zb = zv[pl.ds(base, LANES)]
            jv = base + lane_i
            new_offs = []
            for q in range(P):
                dx = xb - xi[q]
                dy = yb - yi[q]
                dz = zb - zi[q]
                d2 = dx * dx + dy * dy + dz * dz
                m = d2 < R2
                incl = plsc.cumsum(m.astype(jnp.int32))
                idx = offs[q] + incl
                plsc.store_scatter(nbufs[q], [idx], jv, mask=m)
                new_offs.append(offs[q] + incl[LANES - 1])
            return tuple(new_offs)

        moff = jnp.full((LANES,), -1, jnp.int32)
        cntv = lax.fori_loop(0, NBLK, blk, (moff,) * P)
        cnts = [cntv[q][0] + 1 for q in range(P)]

        for q in range(P):
            ip = ipass * P + q
            nbq = nbufs[q]
            cnt = cnts[q]

            def b1(mb, _):
                o = mb * LANES
                jvq = nbq[pl.ds(o, LANES)]
                dx = plsc.load_gather(xv, [jvq]) - xi[q]
                dy = plsc.load_gather(yv, [jvq]) - yi[q]
                dz = plsc.load_gather(zv, [jvq]) - zi[q]
                d2v = dx * dx + dy * dy + dz * dz + 1e-12
                iv = plsc.bitcast(d2v, jnp.int32)
                iv = 0x5F3759DF - (iv >> 1)
                z = plsc.bitcast(iv, jnp.float32)
                z = z * (1.5 - 0.5 * d2v * z * z)
                z = z * (1.5 - 0.5 * d2v * z * z)
                z = z * (1.5 - 0.5 * d2v * z * z)
                ubuf[pl.ds(o, LANES)] = GAMMA * (d2v * z)
                return 0

            lax.fori_loop(0, (cnt + LANES - 1) // LANES, b1, 0)

            def b2(m, accs):
                j = nbq[pl.ds(m, LANES)][0]
                u = ubuf[pl.ds(m, LANES)][0]
                e = u - kvec
                rb = jnp.exp(-(e * e))
                fidx = j * C_IN + lane_i
                f0 = plsc.load_gather(fv, [fidx])
                f1 = plsc.load_gather(fv, [fidx + 16])
                out = []
                for k in range(NUM_BASIS):
                    w = rb[k]
                    out.append(accs[2 * k] + w * f0)
                    out.append(accs[2 * k + 1] + w * f1)
                return tuple(out)

            accs = lax.fori_loop(0, cnt, b2, (zero16,) * (2 * NUM_BASIS))

            rowbase = (ip % HALF) * ROW
            for k in range(NUM_BASIS):
                stage[pl.ds(rowbase + 32 * k, 16)] = accs[2 * k]
                stage[pl.ds(rowbase + 32 * k + 16, 16)] = accs[2 * k + 1]
            for k in range(NUM_BASIS, KPAD):
                stage[pl.ds(rowbase + 32 * k, 16)] = zero16
                stage[pl.ds(rowbase + 32 * k + 16, 16)] = zero16
        return 0

    out_base = b * (N * ROW) + s * PTS * ROW
    lax.fori_loop(0, HALF // P, pass_body, 0)
    pltpu.sync_copy(stage, acc_out.at[pl.ds(out_base, HALF * ROW)])
    lax.fori_loop(HALF // P, PTS // P, pass_body, 0)
    pltpu.sync_copy(
        stage, acc_out.at[pl.ds(out_base + HALF * ROW, HALF * ROW)])


def _tc_body(acc_ref, wt_ref, out_ref):
    out_ref[0] = lax.dot_general(
        wt_ref[...], acc_ref[0],
        (((1,), (1,)), ((), ())),
        preferred_element_type=jnp.float32,
    )


def kernel(features, geometry, W):
    featF = jnp.transpose(features, (0, 2, 1)).reshape(BATCH * N * C_IN)
    geomT = jnp.transpose(geometry, (0, 2, 1)).reshape(BATCH * 3 * N)
    wt = jnp.transpose(W, (1, 0, 2)).reshape(C_OUT, NUM_BASIS * C_IN)
    wt = jnp.pad(wt, ((0, 0), (0, (KPAD - NUM_BASIS) * C_IN)))

    mesh = plsc.VectorSubcoreMesh(core_axis_name="c", subcore_axis_name="s")
    sc = pl.kernel(
        _sc_body,
        mesh=mesh,
        compiler_params=pltpu.CompilerParams(needs_layout_passes=False),
        out_type=jax.ShapeDtypeStruct((BATCH * N * ROW,), jnp.float32),
        scratch_types=[
            pltpu.VMEM((N,), jnp.float32),
            pltpu.VMEM((N,), jnp.float32),
            pltpu.VMEM((N,), jnp.float32),
            pltpu.VMEM((N * C_IN,), jnp.float32),
            pltpu.VMEM((N + LANES,), jnp.int32),
            pltpu.VMEM((N + LANES,), jnp.int32),
            pltpu.VMEM((N + LANES,), jnp.int32),
            pltpu.VMEM((N + LANES,), jnp.int32),
            pltpu.VMEM((N + LANES,), jnp.float32),
            pltpu.VMEM((HALF * ROW,), jnp.float32),
        ],
    )
    acc = sc(geomT, featF).reshape(BATCH, N, ROW)

    nt = 8
    out = pl.pallas_call(
        _tc_body,
        grid=(BATCH, nt),
        in_specs=[
            pl.BlockSpec((1, N // nt, ROW), lambda b, t: (b, t, 0)),
            pl.BlockSpec((C_OUT, ROW), lambda b, t: (0, 0)),
        ],
        out_specs=pl.BlockSpec((1, C_OUT, N // nt), lambda b, t: (b, 0, t)),
        out_shape=jax.ShapeDtypeStruct((BATCH, C_OUT, N), jnp.float32),
    )(acc, wt)
    return out


# speedup vs baseline: 1.1598x; 1.0000x over previous
import jax
import jax.numpy as jnp
from jax import lax
from jax.experimental import pallas as pl
from jax.experimental.pallas import tpu as pltpu
from jax.experimental.pallas import tpu_sc as plsc

RADIUS = 0.4
NUM_BASIS = 10
C_IN = 32
C_OUT = 32
GAMMA = (NUM_BASIS - 1) / RADIUS

N = 2048
BATCH = 2
NCORES = 2
NSUB = 16
LANES = 16
PTS = N // NSUB
NBLK = N // LANES
KPAD = 12
ROW = KPAD * C_IN
R2 = RADIUS * RADIUS


P = 4
HALF = PTS // 2


def _sc_body(geom, featT, acc_out, xv, yv, zv, fv, nb0, nb1, nb2, nb3, ubuf,
             stage):
    b = lax.axis_index("c")
    s = lax.axis_index("s")
    gbase = b * (3 * N)
    pltpu.sync_copy(geom.at[pl.ds(gbase, N)], xv)
    pltpu.sync_copy(geom.at[pl.ds(gbase + N, N)], yv)
    pltpu.sync_copy(geom.at[pl.ds(gbase + 2 * N, N)], zv)
    pltpu.sync_copy(featT.at[pl.ds(b * (N * C_IN), N * C_IN)], fv)

    lane_i = lax.iota(jnp.int32, 16)
    kvec = lane_i.astype(jnp.float32)
    zero16 = jnp.zeros((16,), jnp.float32)
    zero16i = jnp.zeros((16,), jnp.int32)
    nbufs = (nb0, nb1, nb2, nb3)

    def zinit(i, _):
        for nb in nbufs:
            nb[pl.ds(i * LANES, LANES)] = zero16i
        return 0

    lax.fori_loop(0, (N + LANES) // LANES, zinit, 0)

    def pass_body(ipass, _):
        p0 = s * PTS + ipass * P
        xi, yi, zi = [], [], []
        for q in range(P):
            pidx = jnp.full((LANES,), p0 + q, jnp.int32)
            xi.append(plsc.load_gather(xv, [pidx]))
            yi.append(plsc.load_gather(yv, [pidx]))
            zi.append(plsc.load_gather(zv, [pidx]))

        def blk(jb, offs):
            base = jb * LANES
            xb = xv[pl.ds(base, LANES)]
            yb = yv[pl.ds(base, LANES)]
            zb = zv[pl.ds(base, LANES)]
            jv = base + lane_i
            new_offs = []
            for q in range(P):
                dx = xb - xi[q]
                dy = yb - yi[q]
                dz = zb - zi[q]
                d2 = dx * dx + dy * dy + dz * dz
                m = d2 < R2
                incl = plsc.cumsum(m.astype(jnp.int32))
                idx = offs[q] + incl
                plsc.store_scatter(nbufs[q], [idx], jv, mask=m)
                new_offs.append(offs[q] + incl[LANES - 1])
            return tuple(new_offs)

        moff = jnp.full((LANES,), -1, jnp.int32)
        cntv = lax.fori_loop(0, NBLK, blk, (moff,) * P)
        cnts = [cntv[q][0] + 1 for q in range(P)]

        for q in range(P):
            ip = ipass * P + q
            nbq = nbufs[q]
            cnt = cnts[q]

            def b1(mb, _):
                o = mb * LANES
                jvq = nbq[pl.ds(o, LANES)]
                dx = plsc.load_gather(xv, [jvq]) - xi[q]
                dy = plsc.load_gather(yv, [jvq]) - yi[q]
                dz = plsc.load_gather(zv, [jvq]) - zi[q]
                d2v = dx * dx + dy * dy + dz * dz + 1e-12
                iv = plsc.bitcast(d2v, jnp.int32)
                iv = 0x5F3759DF - (iv >> 1)
                z = plsc.bitcast(iv, jnp.float32)
                z = z * (1.5 - 0.5 * d2v * z * z)
                z = z * (1.5 - 0.5 * d2v * z * z)
                z = z * (1.5 - 0.5 * d2v * z * z)
                ubuf[pl.ds(o, LANES)] = GAMMA * (d2v * z)
                return 0

            lax.fori_loop(0, (cnt + LANES - 1) // LANES, b1, 0)

            def b2(m, accs):
                j = nbq[pl.ds(m, LANES)][0]
                u = ubuf[pl.ds(m, LANES)][0]
                e = u - kvec
                rb = jnp.exp(-(e * e))
                fidx = j * C_IN + lane_i
                f0 = plsc.load_gather(fv, [fidx])
                f1 = plsc.load_gather(fv, [fidx + 16])
                out = []
                for k in range(NUM_BASIS):
                    w = rb[k]
                    out.append(accs[2 * k] + w * f0)
                    out.append(accs[2 * k + 1] + w * f1)
                return tuple(out)

            accs = lax.fori_loop(0, cnt, b2, (zero16,) * (2 * NUM_BASIS))

            rowbase = (ip % HALF) * ROW
            for k in range(NUM_BASIS):
                stage[pl.ds(rowbase + 32 * k, 16)] = accs[2 * k]
                stage[pl.ds(rowbase + 32 * k + 16, 16)] = accs[2 * k + 1]
            for k in range(NUM_BASIS, KPAD):
                stage[pl.ds(rowbase + 32 * k, 16)] = zero16
                stage[pl.ds(rowbase + 32 * k + 16, 16)] = zero16
        return 0

    out_base = b * (N * ROW) + s * PTS * ROW
    lax.fori_loop(0, HALF // P, pass_body, 0)
    pltpu.sync_copy(stage, acc_out.at[pl.ds(out_base, HALF * ROW)])
    lax.fori_loop(HALF // P, PTS // P, pass_body, 0)
    pltpu.sync_copy(
        stage, acc_out.at[pl.ds(out_base + HALF * ROW, HALF * ROW)])


def _tc_body(acc_ref, wt_ref, out_ref):
    out_ref[0] = lax.dot_general(
        wt_ref[...], acc_ref[0],
        (((1,), (1,)), ((), ())),
        preferred_element_type=jnp.float32,
    )


def kernel(features, geometry, W):
    featF = jnp.transpose(features, (0, 2, 1)).reshape(BATCH * N * C_IN)
    geomT = jnp.transpose(geometry, (0, 2, 1)).reshape(BATCH * 3 * N)
    wt = jnp.transpose(W, (1, 0, 2)).reshape(C_OUT, NUM_BASIS * C_IN)
    wt = jnp.pad(wt, ((0, 0), (0, (KPAD - NUM_BASIS) * C_IN)))

    mesh = plsc.VectorSubcoreMesh(core_axis_name="c", subcore_axis_name="s")
    sc = pl.kernel(
        _sc_body,
        mesh=mesh,
        compiler_params=pltpu.CompilerParams(needs_layout_passes=False),
        out_type=jax.ShapeDtypeStruct((BATCH * N * ROW,), jnp.float32),
        scratch_types=[
            pltpu.VMEM((N,), jnp.float32),
            pltpu.VMEM((N,), jnp.float32),
            pltpu.VMEM((N,), jnp.float32),
            pltpu.VMEM((N * C_IN,), jnp.float32),
            pltpu.VMEM((N + LANES,), jnp.int32),
            pltpu.VMEM((N + LANES,), jnp.int32),
            pltpu.VMEM((N + LANES,), jnp.int32),
            pltpu.VMEM((N + LANES,), jnp.int32),
            pltpu.VMEM((N + LANES,), jnp.float32),
            pltpu.VMEM((HALF * ROW,), jnp.float32),
        ],
    )
    acc = sc(geomT, featF).reshape(BATCH, N, ROW)

    nt = 8
    out = pl.pallas_call(
        _tc_body,
        grid=(BATCH, nt),
        in_specs=[
            pl.BlockSpec((1, N // nt, ROW), lambda b, t: (b, t, 0)),
            pl.BlockSpec((C_OUT, ROW), lambda b, t: (0, 0)),
        ],
        out_specs=pl.BlockSpec((1, C_OUT, N // nt), lambda b, t: (b, 0, t)),
        out_shape=jax.ShapeDtypeStruct((BATCH, C_OUT, N), jnp.float32),
    )(acc, wt)
    return out
